# baseline (speedup 1.0000x reference)
import functools

import jax
import jax.numpy as jnp
from jax import lax
from jax.experimental import pallas as pl
from jax.experimental.pallas import tpu as pltpu
from jax.experimental.pallas import tpu_sc as plsc

_N_NODES = 10000
_N_EDGES = 320000
_D = 128
_RBF = 16

_NC = 2
_NS = 16
_NW = _NC * _NS
_CHUNK = 128
_NSLICE = 8
_CPW = 10
_EPW = _CPW * _CHUNK
_SLICE_E = _NW * _EPW
_E_PAD = _NSLICE * _SLICE_E

_BE = 2560

import numpy as _np
_PERM = _np.array(list(range(0, _D, 2)) + list(range(1, _D, 2)))


def _node_proj_body(ns_ref, ws_ref, wd_ref, ps_ref, pd_ref):
    x = ns_ref[...]
    ps_ref[...] = jnp.dot(
        x, ws_ref[...], preferred_element_type=jnp.float32).astype(jnp.bfloat16)
    pd_ref[...] = jnp.dot(
        x, wd_ref[...], preferred_element_type=jnp.float32).astype(jnp.bfloat16)


def _gather_add_body(ps_hbm, pd_hbm, src_hbm, dst_hbm, out_hbm,
                     idx_s, idx_d, r1a, r2a, r1b, r2b, oa, ob,
                     sg1a, sg2a, sg1b, sg2b, sta, stb):
    wid = lax.axis_index("s") * _NC + lax.axis_index("c")
    base0 = wid * _EPW

    pltpu.sync_copy(src_hbm.at[pl.ds(base0, _EPW)], idx_s)
    pltpu.sync_copy(dst_hbm.at[pl.ds(base0, _EPW)], idx_d)

    def issue_gather(t, r1, r2, s1, s2):
        isl = pl.ds(t * _CHUNK, _CHUNK)
        pltpu.async_copy(ps_hbm.at[idx_s.at[isl]], r1, s1)
        pltpu.async_copy(pd_hbm.at[idx_d.at[isl]], r2, s2)

    def wait_gather(r1, r2, s1, s2):
        pltpu.make_async_copy(ps_hbm.at[idx_s.at[pl.ds(0, _CHUNK)]], r1, s1).wait()
        pltpu.make_async_copy(pd_hbm.at[idx_d.at[pl.ds(0, _CHUNK)]], r2, s2).wait()

    def add_rows(r1, r2, o):
        @pl.loop(0, _CHUNK // 2, unroll=2)
        def _row(i):
            for e in range(2):
                for j in range(4):
                    sj = pl.ds(j * 16, 16)
                    a = plsc.bitcast(r1[i + 64 * e, sj], jnp.bfloat16)
                    b = plsc.bitcast(r2[i + 64 * e, sj], jnp.bfloat16)
                    ae, ao = plsc.unpack(a, format=plsc.PackFormat.INTERLEAVED)
                    be, bo = plsc.unpack(b, format=plsc.PackFormat.INTERLEAVED)
                    packed = plsc.pack(ae + be, ao + bo,
                                       format=plsc.PackFormat.INTERLEAVED)
                    o[i, pl.ds(e * 64 + j * 16, 16)] = plsc.bitcast(packed, jnp.int32)

    def issue_store(t, o, sem):
        pltpu.async_copy(
            o, out_hbm.at[pl.ds(base0 // 2 + t * (_CHUNK // 2), _CHUNK // 2)], sem)

    def wait_store(o, sem):
        pltpu.make_async_copy(o, out_hbm.at[pl.ds(0, _CHUNK // 2)], sem).wait()

    issue_gather(0, r1a, r2a, sg1a, sg2a)

    @pl.loop(0, _CPW, step=2)
    def _chunk(t):
        @pl.when(t > 1)
        def _():
            wait_store(oa, sta)
        issue_gather(t + 1, r1b, r2b, sg1b, sg2b)
        wait_gather(r1a, r2a, sg1a, sg2a)
        add_rows(r1a, r2a, oa)
        issue_store(t, oa, sta)

        @pl.when(t > 0)
        def _():
            wait_store(ob, stb)

        @pl.when(t + 2 < _CPW)
        def _():
            issue_gather(t + 2, r1a, r2a, sg1a, sg2a)
        wait_gather(r1b, r2b, sg1b, sg2b)
        add_rows(r1b, r2b, ob)
        issue_store(t + 1, ob, stb)

    wait_store(oa, sta)
    wait_store(ob, stb)


def _mlp_body(g_ref, ef_ref, d_ref, we_ref, wd_ref, w2_ref,
              b1_ref, b2_ref, gam_ref, bet_ref, o_ref):
    ef = ef_ref[...]
    gi = g_ref[...]
    lo = lax.bitcast_convert_type(gi << 16, jnp.float32)
    hi = lax.bitcast_convert_type(gi & jnp.int32(-65536), jnp.float32)
    ga = jnp.concatenate([lo[:, :_D // 2], hi[:, :_D // 2]], axis=1)
    gb = jnp.concatenate([lo[:, _D // 2:], hi[:, _D // 2:]], axis=1)
    nc = ef.shape[0] // _CHUNK
    gx = jnp.concatenate(
        [ga.reshape(nc, _CHUNK // 2, _D), gb.reshape(nc, _CHUNK // 2, _D)],
        axis=1).reshape(ef.shape[0], _D)
    x = gx + jnp.dot(ef, we_ref[...], preferred_element_type=jnp.float32)
    x = x + jnp.dot(d_ref[...], wd_ref[...], preferred_element_type=jnp.float32)
    x = x + b1_ref[...]
    x = x * (1.0 / (1.0 + jnp.exp(-x)))
    y = jnp.dot(x, w2_ref[...], preferred_element_type=jnp.float32) + b2_ref[...]
    y = y * (1.0 / (1.0 + jnp.exp(-y)))
    z = ef + y
    mu = jnp.mean(z, axis=1, keepdims=True)
    zc = z - mu
    var = jnp.mean(zc * zc, axis=1, keepdims=True)
    o_ref[...] = zc * lax.rsqrt(var + 1e-5) * gam_ref[...] + bet_ref[...]


def _mlp_body_acc(acc_ref, g_ref, ef_ref, d_ref, we_ref, wd_ref, w2_ref,
                  b1_ref, b2_ref, gam_ref, bet_ref, o_ref):
    del acc_ref
    _mlp_body(g_ref, ef_ref, d_ref, we_ref, wd_ref, w2_ref,
              b1_ref, b2_ref, gam_ref, bet_ref, o_ref)


def kernel(node_scalars, edge_feats, d, W1, b1, W2, b2, ln_gamma, ln_beta, edge_index):
    idx = edge_index.astype(jnp.int32)
    src = jnp.pad(idx[0], (0, _E_PAD - _N_EDGES))
    dst = jnp.pad(idx[1], (0, _E_PAD - _N_EDGES))

    ps, pd = pl.pallas_call(
        _node_proj_body,
        out_shape=[jax.ShapeDtypeStruct((_N_NODES, _D), jnp.bfloat16)] * 2,
    )(node_scalars, W1[0:_D], W1[_D:2 * _D])

    sc_gather = pl.kernel(
        _gather_add_body,
        out_type=jax.ShapeDtypeStruct((_SLICE_E // 2, _D), jnp.int32),
        mesh=plsc.VectorSubcoreMesh(core_axis_name="c", subcore_axis_name="s"),
        compiler_params=pltpu.CompilerParams(use_tc_tiling_on_sc=False, needs_layout_passes=False),
        scratch_types=[
            pltpu.VMEM((_EPW,), jnp.int32),
            pltpu.VMEM((_EPW,), jnp.int32),
            pltpu.VMEM((_CHUNK, _D // 2), jnp.int32),
            pltpu.VMEM((_CHUNK, _D // 2), jnp.int32),
            pltpu.VMEM((_CHUNK, _D // 2), jnp.int32),
            pltpu.VMEM((_CHUNK, _D // 2), jnp.int32),
            pltpu.VMEM((_CHUNK // 2, _D), jnp.int32),
            pltpu.VMEM((_CHUNK // 2, _D), jnp.int32),
            pltpu.SemaphoreType.DMA,
            pltpu.SemaphoreType.DMA,
            pltpu.SemaphoreType.DMA,
            pltpu.SemaphoreType.DMA,
            pltpu.SemaphoreType.DMA,
            pltpu.SemaphoreType.DMA,
        ],
    )
    ps_i = lax.bitcast_convert_type(ps.reshape(_N_NODES, _D // 2, 2), jnp.int32)
    pd_i = lax.bitcast_convert_type(pd.reshape(_N_NODES, _D // 2, 2), jnp.int32)

    we_p = W1[2 * _D:3 * _D][:, _PERM]
    wd_p = W1[3 * _D:][:, _PERM]
    w2_p = W2[_PERM, :]
    b1_p = b1[_PERM].reshape(1, _D)
    b2_r = b2.reshape(1, _D)
    gam_r = ln_gamma.reshape(1, _D)
    bet_r = ln_beta.reshape(1, _D)
    full = lambda i: (0, 0)

    gs = []
    for sl in range(_NSLICE):
        lo = sl * _SLICE_E
        gs.append(sc_gather(ps_i, pd_i,
                            lax.slice(src, (lo,), (lo + _SLICE_E,)),
                            lax.slice(dst, (lo,), (lo + _SLICE_E,))))

    acc = None
    for sl in range(_NSLICE):
        lo = sl * _SLICE_E
        nb = min(_SLICE_E, _N_EDGES - lo) // _BE
        off = lo // _BE
        specs = [
            pl.BlockSpec((_BE // 2, _D), lambda i: (i, 0)),
            pl.BlockSpec((_BE, _D), lambda i, o=off: (o + i, 0)),
            pl.BlockSpec((_BE, _RBF), lambda i, o=off: (o + i, 0)),
            pl.BlockSpec((_D, _D), full),
            pl.BlockSpec((_RBF, _D), full),
            pl.BlockSpec((_D, _D), full),
            pl.BlockSpec((1, _D), full),
            pl.BlockSpec((1, _D), full),
            pl.BlockSpec((1, _D), full),
            pl.BlockSpec((1, _D), full),
        ]
        args = (gs[sl], edge_feats, d, we_p, wd_p, w2_p, b1_p, b2_r, gam_r, bet_r)
        if sl == 0:
            acc = pl.pallas_call(
                _mlp_body,
                grid=(nb,),
                in_specs=specs,
                out_specs=pl.BlockSpec((_BE, _D), lambda i, o=off: (o + i, 0)),
                out_shape=jax.ShapeDtypeStruct((_N_EDGES, _D), jnp.float32),
            )(*args)
        else:
            acc = pl.pallas_call(
                _mlp_body_acc,
                grid=(nb,),
                in_specs=[pl.BlockSpec(memory_space=pl.ANY)] + specs,
                out_specs=pl.BlockSpec((_BE, _D), lambda i, o=off: (o + i, 0)),
                out_shape=jax.ShapeDtypeStruct((_N_EDGES, _D), jnp.float32),
                input_output_aliases={0: 0},
            )(acc, *args)
    return acc


# speedup vs baseline: 1.0170x; 1.0170x over previous
import functools

import jax
import jax.numpy as jnp
from jax import lax
from jax.experimental import pallas as pl
from jax.experimental.pallas import tpu as pltpu
from jax.experimental.pallas import tpu_sc as plsc

_N_NODES = 10000
_N_EDGES = 320000
_D = 128
_RBF = 16

_NC = 2
_NS = 16
_NW = _NC * _NS
_CHUNK = 128
_SLICE_SIZES = (122880, 81920, 81920, 40960)
_E_PAD = sum(_SLICE_SIZES)

_BE = 2560

import numpy as _np
_PERM = _np.array(list(range(0, _D, 2)) + list(range(1, _D, 2)))


def _node_proj_body(ns_ref, ws_ref, wd_ref, ps_ref, pd_ref):
    x = ns_ref[...]
    ps_ref[...] = jnp.dot(
        x, ws_ref[...], preferred_element_type=jnp.float32).astype(jnp.bfloat16)
    pd_ref[...] = jnp.dot(
        x, wd_ref[...], preferred_element_type=jnp.float32).astype(jnp.bfloat16)


def _make_gather_add_body(cpw):
  epw = cpw * _CHUNK

  def _gather_add_body(ps_hbm, pd_hbm, src_hbm, dst_hbm, out_hbm,
                       idx_s, idx_d, r1a, r2a, r1b, r2b, oa, ob,
                       sg1a, sg2a, sg1b, sg2b, sta, stb):
    wid = lax.axis_index("s") * _NC + lax.axis_index("c")
    base0 = wid * epw

    pltpu.sync_copy(src_hbm.at[pl.ds(base0, epw)], idx_s)
    pltpu.sync_copy(dst_hbm.at[pl.ds(base0, epw)], idx_d)

    def issue_gather(t, r1, r2, s1, s2):
        isl = pl.ds(t * _CHUNK, _CHUNK)
        pltpu.async_copy(ps_hbm.at[idx_s.at[isl]], r1, s1)
        pltpu.async_copy(pd_hbm.at[idx_d.at[isl]], r2, s2)

    def wait_gather(r1, r2, s1, s2):
        pltpu.make_async_copy(ps_hbm.at[idx_s.at[pl.ds(0, _CHUNK)]], r1, s1).wait()
        pltpu.make_async_copy(pd_hbm.at[idx_d.at[pl.ds(0, _CHUNK)]], r2, s2).wait()

    def add_rows(r1, r2, o):
        @pl.loop(0, _CHUNK // 2, unroll=2)
        def _row(i):
            for e in range(2):
                for j in range(4):
                    sj = pl.ds(j * 16, 16)
                    a = plsc.bitcast(r1[i + 64 * e, sj], jnp.bfloat16)
                    b = plsc.bitcast(r2[i + 64 * e, sj], jnp.bfloat16)
                    ae, ao = plsc.unpack(a, format=plsc.PackFormat.INTERLEAVED)
                    be, bo = plsc.unpack(b, format=plsc.PackFormat.INTERLEAVED)
                    packed = plsc.pack(ae + be, ao + bo,
                                       format=plsc.PackFormat.INTERLEAVED)
                    o[i, pl.ds(e * 64 + j * 16, 16)] = plsc.bitcast(packed, jnp.int32)

    def issue_store(t, o, sem):
        pltpu.async_copy(
            o, out_hbm.at[pl.ds(base0 // 2 + t * (_CHUNK // 2), _CHUNK // 2)], sem)

    def wait_store(o, sem):
        pltpu.make_async_copy(o, out_hbm.at[pl.ds(0, _CHUNK // 2)], sem).wait()

    issue_gather(0, r1a, r2a, sg1a, sg2a)

    @pl.loop(0, cpw, step=2)
    def _chunk(t):
        @pl.when(t > 1)
        def _():
            wait_store(oa, sta)
        issue_gather(t + 1, r1b, r2b, sg1b, sg2b)
        wait_gather(r1a, r2a, sg1a, sg2a)
        add_rows(r1a, r2a, oa)
        issue_store(t, oa, sta)

        @pl.when(t > 0)
        def _():
            wait_store(ob, stb)

        @pl.when(t + 2 < cpw)
        def _():
            issue_gather(t + 2, r1a, r2a, sg1a, sg2a)
        wait_gather(r1b, r2b, sg1b, sg2b)
        add_rows(r1b, r2b, ob)
        issue_store(t + 1, ob, stb)

    wait_store(oa, sta)
    wait_store(ob, stb)

  return _gather_add_body


def _mlp_body(g_ref, ef_ref, d_ref, we_ref, wd_ref, w2_ref,
              b1_ref, b2_ref, gam_ref, bet_ref, o_ref):
    ef = ef_ref[...]
    gi = g_ref[...]
    lo = lax.bitcast_convert_type(gi << 16, jnp.float32)
    hi = lax.bitcast_convert_type(gi & jnp.int32(-65536), jnp.float32)
    ga = jnp.concatenate([lo[:, :_D // 2], hi[:, :_D // 2]], axis=1)
    gb = jnp.concatenate([lo[:, _D // 2:], hi[:, _D // 2:]], axis=1)
    nc = ef.shape[0] // _CHUNK
    gx = jnp.concatenate(
        [ga.reshape(nc, _CHUNK // 2, _D), gb.reshape(nc, _CHUNK // 2, _D)],
        axis=1).reshape(ef.shape[0], _D)
    x = gx + jnp.dot(ef, we_ref[...], preferred_element_type=jnp.float32)
    x = x + jnp.dot(d_ref[...], wd_ref[...], preferred_element_type=jnp.float32)
    x = x + b1_ref[...]
    x = x * (1.0 / (1.0 + jnp.exp(-x)))
    y = jnp.dot(x, w2_ref[...], preferred_element_type=jnp.float32) + b2_ref[...]
    y = y * (1.0 / (1.0 + jnp.exp(-y)))
    z = ef + y
    mu = jnp.mean(z, axis=1, keepdims=True)
    zc = z - mu
    var = jnp.mean(zc * zc, axis=1, keepdims=True)
    o_ref[...] = zc * lax.rsqrt(var + 1e-5) * gam_ref[...] + bet_ref[...]


def _mlp_body_acc(acc_ref, g_ref, ef_ref, d_ref, we_ref, wd_ref, w2_ref,
                  b1_ref, b2_ref, gam_ref, bet_ref, o_ref):
    del acc_ref
    _mlp_body(g_ref, ef_ref, d_ref, we_ref, wd_ref, w2_ref,
              b1_ref, b2_ref, gam_ref, bet_ref, o_ref)


def kernel(node_scalars, edge_feats, d, W1, b1, W2, b2, ln_gamma, ln_beta, edge_index):
    idx = edge_index.astype(jnp.int32)
    src = jnp.pad(idx[0], (0, _E_PAD - _N_EDGES))
    dst = jnp.pad(idx[1], (0, _E_PAD - _N_EDGES))

    ps, pd = pl.pallas_call(
        _node_proj_body,
        out_shape=[jax.ShapeDtypeStruct((_N_NODES, _D), jnp.bfloat16)] * 2,
    )(node_scalars, W1[0:_D], W1[_D:2 * _D])

    def make_sc_gather(size):
      cpw = size // (_NW * _CHUNK)
      return pl.kernel(
        _make_gather_add_body(cpw),
        out_type=jax.ShapeDtypeStruct((size // 2, _D), jnp.int32),
        mesh=plsc.VectorSubcoreMesh(core_axis_name="c", subcore_axis_name="s"),
        compiler_params=pltpu.CompilerParams(use_tc_tiling_on_sc=False, needs_layout_passes=False),
        scratch_types=[
            pltpu.VMEM((cpw * _CHUNK,), jnp.int32),
            pltpu.VMEM((cpw * _CHUNK,), jnp.int32),
            pltpu.VMEM((_CHUNK, _D // 2), jnp.int32),
            pltpu.VMEM((_CHUNK, _D // 2), jnp.int32),
            pltpu.VMEM((_CHUNK, _D // 2), jnp.int32),
            pltpu.VMEM((_CHUNK, _D // 2), jnp.int32),
            pltpu.VMEM((_CHUNK // 2, _D), jnp.int32),
            pltpu.VMEM((_CHUNK // 2, _D), jnp.int32),
            pltpu.SemaphoreType.DMA,
            pltpu.SemaphoreType.DMA,
            pltpu.SemaphoreType.DMA,
            pltpu.SemaphoreType.DMA,
            pltpu.SemaphoreType.DMA,
            pltpu.SemaphoreType.DMA,
        ],
    )
    ps_i = lax.bitcast_convert_type(ps.reshape(_N_NODES, _D // 2, 2), jnp.int32)
    pd_i = lax.bitcast_convert_type(pd.reshape(_N_NODES, _D // 2, 2), jnp.int32)

    we_p = W1[2 * _D:3 * _D][:, _PERM]
    wd_p = W1[3 * _D:][:, _PERM]
    w2_p = W2[_PERM, :]
    b1_p = b1[_PERM].reshape(1, _D)
    b2_r = b2.reshape(1, _D)
    gam_r = ln_gamma.reshape(1, _D)
    bet_r = ln_beta.reshape(1, _D)
    full = lambda i: (0, 0)

    gs = []
    offs = [sum(_SLICE_SIZES[:k]) for k in range(len(_SLICE_SIZES))]
    for sl, size in enumerate(_SLICE_SIZES):
        lo = offs[sl]
        gs.append(make_sc_gather(size)(
            ps_i, pd_i,
            lax.slice(src, (lo,), (lo + size,)),
            lax.slice(dst, (lo,), (lo + size,))))

    acc = None
    for sl, size in enumerate(_SLICE_SIZES):
        lo = offs[sl]
        nb = min(size, _N_EDGES - lo) // _BE
        off = lo // _BE
        specs = [
            pl.BlockSpec((_BE // 2, _D), lambda i: (i, 0)),
            pl.BlockSpec((_BE, _D), lambda i, o=off: (o + i, 0)),
            pl.BlockSpec((_BE, _RBF), lambda i, o=off: (o + i, 0)),
            pl.BlockSpec((_D, _D), full),
            pl.BlockSpec((_RBF, _D), full),
            pl.BlockSpec((_D, _D), full),
            pl.BlockSpec((1, _D), full),
            pl.BlockSpec((1, _D), full),
            pl.BlockSpec((1, _D), full),
            pl.BlockSpec((1, _D), full),
        ]
        args = (gs[sl], edge_feats, d, we_p, wd_p, w2_p, b1_p, b2_r, gam_r, bet_r)
        if sl == 0:
            acc = pl.pallas_call(
                _mlp_body,
                grid=(nb,),
                in_specs=specs,
                out_specs=pl.BlockSpec((_BE, _D), lambda i, o=off: (o + i, 0)),
                out_shape=jax.ShapeDtypeStruct((_N_EDGES, _D), jnp.float32),
            )(*args)
        else:
            acc = pl.pallas_call(
                _mlp_body_acc,
                grid=(nb,),
                in_specs=[pl.BlockSpec(memory_space=pl.ANY)] + specs,
                out_specs=pl.BlockSpec((_BE, _D), lambda i, o=off: (o + i, 0)),
                out_shape=jax.ShapeDtypeStruct((_N_EDGES, _D), jnp.float32),
                input_output_aliases={0: 0},
            )(acc, *args)
    return acc
